# baseline (speedup 1.0000x reference)
import jax
import jax.numpy as jnp
from jax.experimental import pallas as pl
from jax.experimental.pallas import tpu as pltpu

B = 4
N = 4096
D = 16
ROW_TILE = 2048
NT = N // ROW_TILE
_INV = 1.0 / (B * N)


def _chamfer_kernel(a_ref, b_ref, out_ref, colmin_ref, yaug_ref):
    bi = pl.program_id(0)
    ti = pl.program_id(1)

    x = a_ref[0]

    def _build_yaug(batch, slot):
        yt = b_ref[batch].T
        yy = jnp.sum(yt * yt, axis=0, keepdims=True)
        ones = jnp.ones((1, N), jnp.float32)
        yaug_ref[slot] = jnp.concatenate(
            [yt * -2.0, ones, yy], axis=0).astype(jnp.bfloat16)

    @pl.when(jnp.logical_and(bi == 0, ti == 0))
    def _build_first():
        _build_yaug(0, 0)

    @pl.when(jnp.logical_and(ti == NT - 1, bi < B - 1))
    def _build_next():
        _build_yaug(bi + 1, (bi + 1) % 2)

    xx = jnp.sum(x * x, axis=1, keepdims=True)
    x_aug = jnp.concatenate(
        [x, xx, jnp.ones((ROW_TILE, 1), jnp.float32)], axis=1
    ).astype(jnp.bfloat16)
    p = jax.lax.dot_general(
        x_aug, yaug_ref[bi % 2],
        dimension_numbers=(((1,), (0,)), ((), ())),
        preferred_element_type=jnp.float32,
    )
    p_bf = p.astype(jnp.bfloat16)
    row_min = jnp.min(p_bf, axis=1).astype(jnp.float32)
    col_min = jnp.min(p_bf, axis=0, keepdims=True)

    @pl.when(jnp.logical_and(bi == 0, ti == 0))
    def _init():
        out_ref[0, 0] = 0.0

    out_ref[0, 0] += jnp.sum(row_min) * _INV

    @pl.when(ti == 0)
    def _col_first():
        colmin_ref[...] = col_min

    @pl.when(ti != 0)
    def _col_rest():
        colmin_ref[...] = jnp.minimum(colmin_ref[...], col_min)

    @pl.when(ti == NT - 1)
    def _col_finish():
        out_ref[0, 0] += jnp.sum(
            colmin_ref[...].astype(jnp.float32)) * _INV


@jax.jit
def kernel(a, b):
    out = pl.pallas_call(
        _chamfer_kernel,
        grid=(B, NT),
        in_specs=[
            pl.BlockSpec((1, ROW_TILE, D), lambda bi, ti: (bi, ti, ti - ti)),
            pl.BlockSpec(
                (B, N, D), lambda bi, ti: (ti - ti, ti - ti, ti - ti)),
        ],
        out_specs=pl.BlockSpec(
            (1, 1), lambda bi, ti: (ti - ti, ti - ti),
            memory_space=pltpu.SMEM,
        ),
        out_shape=jax.ShapeDtypeStruct((1, 1), jnp.float32),
        scratch_shapes=[
            pltpu.VMEM((1, N), jnp.bfloat16),
            pltpu.VMEM((2, D + 2, N), jnp.bfloat16),
        ],
        compiler_params=pltpu.CompilerParams(
            dimension_semantics=("arbitrary", "arbitrary"),
        ),
    )(a, b)
    return out[0, 0]


# speedup vs baseline: 1.0302x; 1.0302x over previous
import jax
import jax.numpy as jnp
from jax.experimental import pallas as pl
from jax.experimental.pallas import tpu as pltpu

B = 4
N = 4096
D = 16
ROW_TILE = 2048
NT = N // ROW_TILE
_INV = 1.0 / (B * N)


def _chamfer_kernel(a_ref, b_ref, out_ref, colmin_ref, yaug_ref):
    bi = pl.program_id(0)
    ti = pl.program_id(1)

    x = a_ref[0]

    @pl.when(ti == 0)
    def _build_yaug():
        yt = b_ref[0].T
        yy = jnp.sum(yt * yt, axis=0, keepdims=True)
        ones = jnp.ones((1, N), jnp.float32)
        yaug_ref[...] = jnp.concatenate(
            [yt, ones, yy], axis=0).astype(jnp.bfloat16)

    xx = jnp.sum(x * x, axis=1, keepdims=True)
    x_aug = jnp.concatenate(
        [x * -2.0, xx, jnp.ones((ROW_TILE, 1), jnp.float32)], axis=1
    ).astype(jnp.bfloat16)
    p = jax.lax.dot_general(
        x_aug, yaug_ref[...],
        dimension_numbers=(((1,), (0,)), ((), ())),
        preferred_element_type=jnp.float32,
    )
    p_bf = p.astype(jnp.bfloat16)
    row_min = jnp.min(p_bf, axis=1).astype(jnp.float32)
    col_min = jnp.min(p_bf, axis=0, keepdims=True)

    @pl.when(jnp.logical_and(bi == 0, ti == 0))
    def _init():
        out_ref[0, 0] = 0.0

    out_ref[0, 0] += jnp.sum(row_min) * _INV

    @pl.when(ti == 0)
    def _col_first():
        colmin_ref[...] = col_min

    @pl.when(ti != 0)
    def _col_rest():
        colmin_ref[...] = jnp.minimum(colmin_ref[...], col_min)

    @pl.when(ti == NT - 1)
    def _col_finish():
        out_ref[0, 0] += jnp.sum(
            colmin_ref[...].astype(jnp.float32)) * _INV


@jax.jit
def kernel(a, b):
    out = pl.pallas_call(
        _chamfer_kernel,
        grid=(B, NT),
        in_specs=[
            pl.BlockSpec((1, ROW_TILE, D), lambda bi, ti: (bi, ti, ti - ti)),
            pl.BlockSpec((1, N, D), lambda bi, ti: (bi, ti - ti, ti - ti)),
        ],
        out_specs=pl.BlockSpec(
            (1, 1), lambda bi, ti: (ti - ti, ti - ti),
            memory_space=pltpu.SMEM,
        ),
        out_shape=jax.ShapeDtypeStruct((1, 1), jnp.float32),
        scratch_shapes=[
            pltpu.VMEM((1, N), jnp.bfloat16),
            pltpu.VMEM((D + 2, N), jnp.bfloat16),
        ],
        compiler_params=pltpu.CompilerParams(
            dimension_semantics=("arbitrary", "arbitrary"),
        ),
    )(a, b)
    return out[0, 0]
